# baseline (speedup 1.0000x reference)
import jax
import jax.numpy as jnp
from jax import lax
from jax.experimental import pallas as pl
from jax.experimental.pallas import tpu as pltpu
from jax.experimental.pallas import tpu_sc as plsc

N_NODE = 10000
NRATE = 5
DIN = 128
DOUT = 32
NE = 320000

NC = 2
NS = 16
NW = NC * NS
EPT = 10240
EPAD = NW * EPT
CPW = EPT // 128
NCHUNK = EPAD // 128
NBAND = 8
FROWS = NBAND * N_NODE
HALF = N_NODE // 2
REGB = 5008
TRASH_BASE = REGB + 1256
ARENA = 7296
ASL = ARENA // NS
DT = 50176
DSL = DT // NS
SBLK = 1000
NBLK = N_NODE // SBLK

_mesh = plsc.VectorSubcoreMesh(core_axis_name="c", subcore_axis_name="s")


def _counts_body(src_h, dst_h, rate_h, g0_out, a0_out, g1_out, a1_out,
                 dsrc_out, ddst_out,
                 srcv, dstv, ratev, g20, a20, g21, a21, ds2, dd2, onesv,
                 zbuf, dsrc_sh, ddst_sh):
    c = lax.axis_index("c")
    s = lax.axis_index("s")
    wid = c * NS + s
    z16 = jnp.zeros((16,), jnp.float32)

    def fz(i, _):
        zbuf[pl.ds(i * 16, 16)] = z16
        return 0
    lax.fori_loop(0, DSL // 16, fz, 0)

    o16 = jnp.ones((16,), jnp.float32)

    def fo(i, _):
        onesv[pl.ds(i * 16, 16)] = o16
        return 0
    lax.fori_loop(0, 8, fo, 0)

    off = pl.multiple_of(s * DSL, 8)
    pltpu.sync_copy(zbuf, dsrc_sh.at[pl.ds(off, DSL)])
    pltpu.sync_copy(zbuf, ddst_sh.at[pl.ds(off, DSL)])
    plsc.subcore_barrier()

    ebase = wid * EPT
    iota16 = lax.iota(jnp.int32, 16)

    def outer(oc, _):
        base = pl.multiple_of(ebase + oc * 1024, 8)
        pltpu.sync_copy(src_h.at[pl.ds(base, 1024)], srcv)
        pltpu.sync_copy(dst_h.at[pl.ds(base, 1024)], dstv)
        pltpu.sync_copy(rate_h.at[pl.ds(base, 1024)], ratev)

        def row(jr, _):
            def col(jc, _):
                o = jr * 128 + jc * 16
                rv = ratev[pl.ds(o, 16)]
                sv = srcv[pl.ds(o, 16)]
                dv = dstv[pl.ds(o, 16)]
                t = jnp.bitwise_and(dv, 3)
                half = jnp.where(dv >= HALF, 1, 0)
                dloc = dv - half * HALF
                band = jnp.minimum(rv, 4) + jnp.where(rv >= 4, t, 0)
                g_raw = band * N_NODE + sv
                a_loc = jnp.where(rv == 4, REGB + jnp.right_shift(dloc, 2),
                                  dloc)
                spread = TRASH_BASE + jnp.bitwise_and(sv + iota16, 1023)
                valid = rv < NRATE
                in0 = valid & (half == 0)
                in1 = valid & (half == 1)
                cs = pl.ds(jc * 16, 16)
                g20[jr, cs] = jnp.where(in0, g_raw, 0)
                a20[jr, cs] = jnp.where(in0, a_loc, spread)
                g21[jr, cs] = jnp.where(in1, g_raw, 0)
                a21[jr, cs] = jnp.where(in1, a_loc, spread)
                dspread = jnp.bitwise_and(iota16 * 11 + sv + dv, 127)
                ds2[jr, cs] = jnp.where(valid, rv * N_NODE + sv,
                                        NRATE * N_NODE + dspread)
                dd2[jr, cs] = jnp.where(valid, rv * N_NODE + dv,
                                        NRATE * N_NODE + dspread)
                return 0
            lax.fori_loop(0, 8, col, 0)
            return 0
        lax.fori_loop(0, 8, row, 0)

        cb = pl.multiple_of(base // 128, 8)
        pltpu.sync_copy(g20, g0_out.at[pl.ds(cb, 8)])
        pltpu.sync_copy(a20, a0_out.at[pl.ds(cb, 8)])
        pltpu.sync_copy(g21, g1_out.at[pl.ds(cb, 8)])
        pltpu.sync_copy(a21, a1_out.at[pl.ds(cb, 8)])

        def sc8(j, _):
            pltpu.sync_copy(onesv, dsrc_sh.at[ds2.at[j]], add=True)
            pltpu.sync_copy(onesv, ddst_sh.at[dd2.at[j]], add=True)
            return 0
        lax.fori_loop(0, 8, sc8, 0)
        return 0
    lax.fori_loop(0, CPW // 8, outer, 0)

    plsc.subcore_barrier()
    coff = pl.multiple_of(c * DT + s * DSL, 8)
    pltpu.sync_copy(dsrc_sh.at[pl.ds(off, DSL)], zbuf)
    pltpu.sync_copy(zbuf, dsrc_out.at[pl.ds(coff, DSL)])
    pltpu.sync_copy(ddst_sh.at[pl.ds(off, DSL)], zbuf)
    pltpu.sync_copy(zbuf, ddst_out.at[pl.ds(coff, DSL)])


_counts_call = pl.kernel(
    _counts_body,
    out_type=[
        jax.ShapeDtypeStruct((NCHUNK, 128), jnp.int32),
        jax.ShapeDtypeStruct((NCHUNK, 128), jnp.int32),
        jax.ShapeDtypeStruct((NCHUNK, 128), jnp.int32),
        jax.ShapeDtypeStruct((NCHUNK, 128), jnp.int32),
        jax.ShapeDtypeStruct((NC * DT,), jnp.float32),
        jax.ShapeDtypeStruct((NC * DT,), jnp.float32),
    ],
    mesh=_mesh,
    scratch_types=[
        pltpu.VMEM((1024,), jnp.int32),
        pltpu.VMEM((1024,), jnp.int32),
        pltpu.VMEM((1024,), jnp.int32),
        pltpu.VMEM((8, 128), jnp.int32),
        pltpu.VMEM((8, 128), jnp.int32),
        pltpu.VMEM((8, 128), jnp.int32),
        pltpu.VMEM((8, 128), jnp.int32),
        pltpu.VMEM((8, 128), jnp.int32),
        pltpu.VMEM((8, 128), jnp.int32),
        pltpu.VMEM((128,), jnp.float32),
        pltpu.VMEM((DSL,), jnp.float32),
        pltpu.VMEM_SHARED((DT,), jnp.float32),
        pltpu.VMEM_SHARED((DT,), jnp.float32),
    ],
)


def _edge_body(feat_h, g0_h, a0_h, g1_h, a1_h, agg_out,
               g2, a2, rows128, zrow, sem, agg_sh):
    c = lax.axis_index("c")
    s = lax.axis_index("s")
    wid = c * NS + s
    z16 = jnp.zeros((16,), jnp.float32)
    cbase = wid * CPW

    for p, (g_h, a_h) in enumerate(((g0_h, a0_h), (g1_h, a1_h))):
        def fz(i, _):
            def fcol(k, _):
                zrow[i, pl.ds(k * 16, 16)] = z16
                return 0
            lax.fori_loop(0, 8, fcol, 0)
            return 0
        lax.fori_loop(0, 152, fz, 0)

        def zc(i, _):
            pltpu.sync_copy(
                zrow,
                agg_sh.at[pl.ds(pl.multiple_of(s * ASL + i * 152, 8), 152)])
            return 0
        lax.fori_loop(0, ASL // 152, zc, 0)
        plsc.subcore_barrier()

        def outer(oc, _):
            cb = pl.multiple_of(cbase + oc * 8, 8)
            pltpu.sync_copy(g_h.at[pl.ds(cb, 8)], g2)
            pltpu.sync_copy(a_h.at[pl.ds(cb, 8)], a2)

            def j8(j, _):
                pltpu.async_copy(feat_h.at[g2.at[j]], rows128, sem).wait()
                pltpu.sync_copy(rows128, agg_sh.at[a2.at[j]], add=True)
                return 0
            lax.fori_loop(0, 8, j8, 0)
            return 0
        lax.fori_loop(0, CPW // 8, outer, 0)

        plsc.subcore_barrier()

        def oc4(i, _):
            soff = pl.multiple_of(s * ASL + i * 152, 8)
            pltpu.sync_copy(agg_sh.at[pl.ds(soff, 152)], zrow)
            pltpu.sync_copy(zrow, agg_out.at[c, p, pl.ds(soff, 152)])
            return 0
        lax.fori_loop(0, ASL // 152, oc4, 0)
    plsc.subcore_barrier()


_edge_call = pl.kernel(
    _edge_body,
    out_type=[jax.ShapeDtypeStruct((NC, 2, ARENA, 128), jnp.float32)],
    mesh=_mesh,
    scratch_types=[
        pltpu.VMEM((8, 128), jnp.int32),
        pltpu.VMEM((8, 128), jnp.int32),
        pltpu.VMEM((128, 128), jnp.float32),
        pltpu.VMEM((152, 128), jnp.float32),
        pltpu.SemaphoreType.DMA,
        pltpu.VMEM_SHARED((ARENA, 128), jnp.float32),
    ],
)


def _feat_body(x_ref, w_ref, dg_ref, o_ref):
    bb = pl.program_id(0)
    slot32 = jnp.where(bb < 4, bb, bb - 4) * DOUT
    deg = dg_ref[0, 0] + dg_ref[1, 0]
    norm = lax.rsqrt(jnp.maximum(deg, 1.0))
    y = jnp.dot(x_ref[...], w_ref[0], preferred_element_type=jnp.float32)
    y = y * norm
    y4 = jnp.concatenate([y, y, y, y], axis=1)
    col = lax.broadcasted_iota(jnp.int32, (SBLK, 128), 1)
    o_ref[...] = jnp.where((col >= slot32) & (col < slot32 + DOUT), y4, 0.0)


_feat_call = pl.pallas_call(
    _feat_body,
    grid=(NBAND, NBLK),
    in_specs=[
        pl.BlockSpec((SBLK, DIN), lambda bb, sb: (sb, 0)),
        pl.BlockSpec((1, DIN, DOUT), lambda bb, sb: (jnp.minimum(bb, 4), 0, 0)),
        pl.BlockSpec((NC, 1, SBLK, 1),
                     lambda bb, sb: (0, jnp.minimum(bb, 4), sb, 0)),
    ],
    out_specs=pl.BlockSpec((SBLK, 128), lambda bb, sb: (bb * NBLK + sb, 0)),
    out_shape=jax.ShapeDtypeStruct((FROWS, 128), jnp.float32),
)


def _epi_body(aggA_ref, aggB_ref, dd_ref, b_ref, o_ref):
    hs = []
    for r in range(NRATE):
        if r < 4:
            agg = aggA_ref[0, :, r] + aggA_ref[1, :, r]
        else:
            agg = aggB_ref[0] + aggB_ref[1]
        deg = dd_ref[0, r] + dd_ref[1, r]
        norm = lax.rsqrt(jnp.maximum(deg, 1.0))
        h = agg * norm + b_ref[r][None, :]
        hs.append(jnp.where(deg > 0.0, h, 0.0))
    o_ref[...] = jnp.concatenate(hs, axis=1)


_epi_call = pl.pallas_call(
    _epi_body,
    grid=(NBLK,),
    in_specs=[
        pl.BlockSpec((NC, SBLK, 4, DOUT), lambda db: (0, db, 0, 0)),
        pl.BlockSpec((NC, SBLK, DOUT), lambda db: (0, db, 0)),
        pl.BlockSpec((NC, NRATE, SBLK, 1), lambda db: (0, 0, db, 0)),
        pl.BlockSpec((NRATE, DOUT), lambda db: (0, 0)),
    ],
    out_specs=pl.BlockSpec((SBLK, NRATE * DOUT), lambda db: (db, 0)),
    out_shape=jax.ShapeDtypeStruct((N_NODE, NRATE * DOUT), jnp.float32),
)


@jax.jit
def kernel(x_src, x_dst, edge_index, rate, W, b):
    src = edge_index[0].astype(jnp.int32)
    dst = edge_index[1].astype(jnp.int32)
    rt = rate.astype(jnp.int32)
    pad = EPAD - NE
    zpad = jnp.zeros((pad,), jnp.int32)
    src_p = jnp.concatenate([src, zpad])
    dst_p = jnp.concatenate([dst, zpad])
    rt_p = jnp.concatenate([rt, jnp.full((pad,), NRATE, jnp.int32)])

    g0, a0, g1, a1, dsrc_p, ddst_p = _counts_call(src_p, dst_p, rt_p)
    dsrc4 = dsrc_p.reshape(NC, DT)[:, :NRATE * N_NODE].reshape(
        NC, NRATE, N_NODE, 1)
    ddst4 = ddst_p.reshape(NC, DT)[:, :NRATE * N_NODE].reshape(
        NC, NRATE, N_NODE, 1)

    feat = _feat_call(x_src, W, dsrc4)
    (arena,) = _edge_call(feat, g0, a0, g1, a1)

    aggA = jnp.concatenate(
        [arena[:, 0, :HALF, :], arena[:, 1, :HALF, :]], axis=1)
    aggA = aggA.reshape(NC, N_NODE, 4, DOUT)
    aggB = jnp.concatenate(
        [arena[:, 0, REGB:REGB + HALF // 4, :],
         arena[:, 1, REGB:REGB + HALF // 4, :]], axis=1)
    aggB = aggB.reshape(NC, N_NODE, DOUT)

    return _epi_call(aggA, aggB, ddst4, b)


# speedup vs baseline: 1.0003x; 1.0003x over previous
import jax
import jax.numpy as jnp
from jax import lax
from jax.experimental import pallas as pl
from jax.experimental.pallas import tpu as pltpu
from jax.experimental.pallas import tpu_sc as plsc

N_NODE = 10000
NRATE = 5
DIN = 128
DOUT = 32
NE = 320000

NC = 2
NS = 16
NW = NC * NS
EPT = 10240
EPAD = NW * EPT
CPW = EPT // 128
NCHUNK = EPAD // 128
NBAND = 8
FROWS = NBAND * N_NODE
HALF = N_NODE // 2
REGB = 5008
TRASH_BASE = REGB + 1256
ARENA = 7296
ASL = ARENA // NS
DT = 50176
DSL = DT // NS
SBLK = 1000
NBLK = N_NODE // SBLK

_mesh = plsc.VectorSubcoreMesh(core_axis_name="c", subcore_axis_name="s")


def _counts_body(src_h, dst_h, rate_h, g0_out, a0_out, g1_out, a1_out,
                 dsrc_out, ddst_out,
                 srcv, dstv, ratev, g20, a20, g21, a21, ds2, dd2, onesv,
                 zbuf, dsrc_sh, ddst_sh):
    c = lax.axis_index("c")
    s = lax.axis_index("s")
    wid = c * NS + s
    z16 = jnp.zeros((16,), jnp.float32)

    def fz(i, _):
        zbuf[pl.ds(i * 16, 16)] = z16
        return 0
    lax.fori_loop(0, DSL // 16, fz, 0)

    o16 = jnp.ones((16,), jnp.float32)

    def fo(i, _):
        onesv[pl.ds(i * 16, 16)] = o16
        return 0
    lax.fori_loop(0, 8, fo, 0)

    off = pl.multiple_of(s * DSL, 8)
    pltpu.sync_copy(zbuf, dsrc_sh.at[pl.ds(off, DSL)])
    pltpu.sync_copy(zbuf, ddst_sh.at[pl.ds(off, DSL)])
    plsc.subcore_barrier()

    ebase = wid * EPT
    iota16 = lax.iota(jnp.int32, 16)

    def outer(oc, _):
        base = pl.multiple_of(ebase + oc * 1024, 8)
        pltpu.sync_copy(src_h.at[pl.ds(base, 1024)], srcv)
        pltpu.sync_copy(dst_h.at[pl.ds(base, 1024)], dstv)
        pltpu.sync_copy(rate_h.at[pl.ds(base, 1024)], ratev)

        def row(jr, _):
            def col(jc, _):
                o = jr * 128 + jc * 16
                rv = ratev[pl.ds(o, 16)]
                sv = srcv[pl.ds(o, 16)]
                dv = dstv[pl.ds(o, 16)]
                t = jnp.bitwise_and(dv, 3)
                half = jnp.where(dv >= HALF, 1, 0)
                dloc = dv - half * HALF
                band = jnp.minimum(rv, 4) + jnp.where(rv >= 4, t, 0)
                g_raw = band * N_NODE + sv
                a_loc = jnp.where(rv == 4, REGB + jnp.right_shift(dloc, 2),
                                  dloc)
                spread = TRASH_BASE + jnp.bitwise_and(sv + iota16, 1023)
                valid = rv < NRATE
                in0 = valid & (half == 0)
                in1 = valid & (half == 1)
                cs = pl.ds(jc * 16, 16)
                g20[jr, cs] = jnp.where(in0, g_raw, 0)
                a20[jr, cs] = jnp.where(in0, a_loc, spread)
                g21[jr, cs] = jnp.where(in1, g_raw, 0)
                a21[jr, cs] = jnp.where(in1, a_loc, spread)
                dspread = jnp.bitwise_and(iota16 * 11 + sv + dv, 127)
                ds2[jr, cs] = jnp.where(valid, rv * N_NODE + sv,
                                        NRATE * N_NODE + dspread)
                dd2[jr, cs] = jnp.where(valid, rv * N_NODE + dv,
                                        NRATE * N_NODE + dspread)
                return 0
            lax.fori_loop(0, 8, col, 0)
            return 0
        lax.fori_loop(0, 8, row, 0)

        cb = pl.multiple_of(base // 128, 8)
        pltpu.sync_copy(g20, g0_out.at[pl.ds(cb, 8)])
        pltpu.sync_copy(a20, a0_out.at[pl.ds(cb, 8)])
        pltpu.sync_copy(g21, g1_out.at[pl.ds(cb, 8)])
        pltpu.sync_copy(a21, a1_out.at[pl.ds(cb, 8)])

        def sc8(j, _):
            pltpu.sync_copy(onesv, dsrc_sh.at[ds2.at[j]], add=True)
            pltpu.sync_copy(onesv, ddst_sh.at[dd2.at[j]], add=True)
            return 0
        lax.fori_loop(0, 8, sc8, 0)
        return 0
    lax.fori_loop(0, CPW // 8, outer, 0)

    plsc.subcore_barrier()
    coff = pl.multiple_of(c * DT + s * DSL, 8)
    pltpu.sync_copy(dsrc_sh.at[pl.ds(off, DSL)], zbuf)
    pltpu.sync_copy(zbuf, dsrc_out.at[pl.ds(coff, DSL)])
    pltpu.sync_copy(ddst_sh.at[pl.ds(off, DSL)], zbuf)
    pltpu.sync_copy(zbuf, ddst_out.at[pl.ds(coff, DSL)])


_counts_call = pl.kernel(
    _counts_body,
    out_type=[
        jax.ShapeDtypeStruct((NCHUNK, 128), jnp.int32),
        jax.ShapeDtypeStruct((NCHUNK, 128), jnp.int32),
        jax.ShapeDtypeStruct((NCHUNK, 128), jnp.int32),
        jax.ShapeDtypeStruct((NCHUNK, 128), jnp.int32),
        jax.ShapeDtypeStruct((NC * DT,), jnp.float32),
        jax.ShapeDtypeStruct((NC * DT,), jnp.float32),
    ],
    mesh=_mesh,
    scratch_types=[
        pltpu.VMEM((1024,), jnp.int32),
        pltpu.VMEM((1024,), jnp.int32),
        pltpu.VMEM((1024,), jnp.int32),
        pltpu.VMEM((8, 128), jnp.int32),
        pltpu.VMEM((8, 128), jnp.int32),
        pltpu.VMEM((8, 128), jnp.int32),
        pltpu.VMEM((8, 128), jnp.int32),
        pltpu.VMEM((8, 128), jnp.int32),
        pltpu.VMEM((8, 128), jnp.int32),
        pltpu.VMEM((128,), jnp.float32),
        pltpu.VMEM((DSL,), jnp.float32),
        pltpu.VMEM_SHARED((DT,), jnp.float32),
        pltpu.VMEM_SHARED((DT,), jnp.float32),
    ],
)


def _edge_body(feat_h, g0_h, a0_h, g1_h, a1_h, agg_out,
               g2, a2, rowsA, rowsB, zrow, semg, sems, agg_sh):
    c = lax.axis_index("c")
    s = lax.axis_index("s")
    wid = c * NS + s
    z16 = jnp.zeros((16,), jnp.float32)
    cbase = wid * CPW

    for p, (g_h, a_h) in enumerate(((g0_h, a0_h), (g1_h, a1_h))):
        def fz(i, _):
            def fcol(k, _):
                zrow[i, pl.ds(k * 16, 16)] = z16
                return 0
            lax.fori_loop(0, 8, fcol, 0)
            return 0
        lax.fori_loop(0, 152, fz, 0)

        def zc(i, _):
            pltpu.sync_copy(
                zrow,
                agg_sh.at[pl.ds(pl.multiple_of(s * ASL + i * 152, 8), 152)])
            return 0
        lax.fori_loop(0, ASL // 152, zc, 0)
        plsc.subcore_barrier()

        def outer(oc, _):
            cb = pl.multiple_of(cbase + oc * 8, 8)
            pltpu.sync_copy(g_h.at[pl.ds(cb, 8)], g2)
            pltpu.sync_copy(a_h.at[pl.ds(cb, 8)], a2)

            g_pend = pltpu.async_copy(feat_h.at[g2.at[0]], rowsA, semg)
            s_pend = None
            for j in range(8):
                cur, nxt = (rowsA, rowsB) if j % 2 == 0 else (rowsB, rowsA)
                g_pend.wait()
                if s_pend is not None:
                    s_pend.wait()
                if j < 7:
                    g_pend = pltpu.async_copy(feat_h.at[g2.at[j + 1]], nxt,
                                              semg)
                s_pend = pltpu.async_copy(cur, agg_sh.at[a2.at[j]], sems,
                                          add=True)
            s_pend.wait()
            return 0
        lax.fori_loop(0, CPW // 8, outer, 0)

        plsc.subcore_barrier()

        def oc4(i, _):
            soff = pl.multiple_of(s * ASL + i * 152, 8)
            pltpu.sync_copy(agg_sh.at[pl.ds(soff, 152)], zrow)
            pltpu.sync_copy(zrow, agg_out.at[c, p, pl.ds(soff, 152)])
            return 0
        lax.fori_loop(0, ASL // 152, oc4, 0)
    plsc.subcore_barrier()


_edge_call = pl.kernel(
    _edge_body,
    out_type=[jax.ShapeDtypeStruct((NC, 2, ARENA, 128), jnp.float32)],
    mesh=_mesh,
    scratch_types=[
        pltpu.VMEM((8, 128), jnp.int32),
        pltpu.VMEM((8, 128), jnp.int32),
        pltpu.VMEM((128, 128), jnp.float32),
        pltpu.VMEM((128, 128), jnp.float32),
        pltpu.VMEM((152, 128), jnp.float32),
        pltpu.SemaphoreType.DMA,
        pltpu.SemaphoreType.DMA,
        pltpu.VMEM_SHARED((ARENA, 128), jnp.float32),
    ],
)


def _feat_body(x_ref, w_ref, dg_ref, o_ref):
    bb = pl.program_id(0)
    slot32 = jnp.where(bb < 4, bb, bb - 4) * DOUT
    deg = dg_ref[0, 0] + dg_ref[1, 0]
    norm = lax.rsqrt(jnp.maximum(deg, 1.0))
    y = jnp.dot(x_ref[...], w_ref[0], preferred_element_type=jnp.float32)
    y = y * norm
    y4 = jnp.concatenate([y, y, y, y], axis=1)
    col = lax.broadcasted_iota(jnp.int32, (SBLK, 128), 1)
    o_ref[...] = jnp.where((col >= slot32) & (col < slot32 + DOUT), y4, 0.0)


_feat_call = pl.pallas_call(
    _feat_body,
    grid=(NBAND, NBLK),
    in_specs=[
        pl.BlockSpec((SBLK, DIN), lambda bb, sb: (sb, 0)),
        pl.BlockSpec((1, DIN, DOUT), lambda bb, sb: (jnp.minimum(bb, 4), 0, 0)),
        pl.BlockSpec((NC, 1, SBLK, 1),
                     lambda bb, sb: (0, jnp.minimum(bb, 4), sb, 0)),
    ],
    out_specs=pl.BlockSpec((SBLK, 128), lambda bb, sb: (bb * NBLK + sb, 0)),
    out_shape=jax.ShapeDtypeStruct((FROWS, 128), jnp.float32),
)


def _epi_body(aggA_ref, aggB_ref, dd_ref, b_ref, o_ref):
    hs = []
    for r in range(NRATE):
        if r < 4:
            agg = aggA_ref[0, :, r] + aggA_ref[1, :, r]
        else:
            agg = aggB_ref[0] + aggB_ref[1]
        deg = dd_ref[0, r] + dd_ref[1, r]
        norm = lax.rsqrt(jnp.maximum(deg, 1.0))
        h = agg * norm + b_ref[r][None, :]
        hs.append(jnp.where(deg > 0.0, h, 0.0))
    o_ref[...] = jnp.concatenate(hs, axis=1)


_epi_call = pl.pallas_call(
    _epi_body,
    grid=(NBLK,),
    in_specs=[
        pl.BlockSpec((NC, SBLK, 4, DOUT), lambda db: (0, db, 0, 0)),
        pl.BlockSpec((NC, SBLK, DOUT), lambda db: (0, db, 0)),
        pl.BlockSpec((NC, NRATE, SBLK, 1), lambda db: (0, 0, db, 0)),
        pl.BlockSpec((NRATE, DOUT), lambda db: (0, 0)),
    ],
    out_specs=pl.BlockSpec((SBLK, NRATE * DOUT), lambda db: (db, 0)),
    out_shape=jax.ShapeDtypeStruct((N_NODE, NRATE * DOUT), jnp.float32),
)


@jax.jit
def kernel(x_src, x_dst, edge_index, rate, W, b):
    src = edge_index[0].astype(jnp.int32)
    dst = edge_index[1].astype(jnp.int32)
    rt = rate.astype(jnp.int32)
    pad = EPAD - NE
    zpad = jnp.zeros((pad,), jnp.int32)
    src_p = jnp.concatenate([src, zpad])
    dst_p = jnp.concatenate([dst, zpad])
    rt_p = jnp.concatenate([rt, jnp.full((pad,), NRATE, jnp.int32)])

    g0, a0, g1, a1, dsrc_p, ddst_p = _counts_call(src_p, dst_p, rt_p)
    dsrc4 = dsrc_p.reshape(NC, DT)[:, :NRATE * N_NODE].reshape(
        NC, NRATE, N_NODE, 1)
    ddst4 = ddst_p.reshape(NC, DT)[:, :NRATE * N_NODE].reshape(
        NC, NRATE, N_NODE, 1)

    feat = _feat_call(x_src, W, dsrc4)
    (arena,) = _edge_call(feat, g0, a0, g1, a1)

    aggA = jnp.concatenate(
        [arena[:, 0, :HALF, :], arena[:, 1, :HALF, :]], axis=1)
    aggA = aggA.reshape(NC, N_NODE, 4, DOUT)
    aggB = jnp.concatenate(
        [arena[:, 0, REGB:REGB + HALF // 4, :],
         arena[:, 1, REGB:REGB + HALF // 4, :]], axis=1)
    aggB = aggB.reshape(NC, N_NODE, DOUT)

    return _epi_call(aggA, aggB, ddst4, b)


# speedup vs baseline: 19.4730x; 19.4671x over previous
import jax
import jax.numpy as jnp
from jax import lax
from jax.experimental import pallas as pl
from jax.experimental.pallas import tpu as pltpu
from jax.experimental.pallas import tpu_sc as plsc

N_NODE = 10000
NRATE = 5
DIN = 128
DOUT = 32
NE = 320000

NC = 2
NS = 16
NW = NC * NS
EPT = 10240
EPAD = NW * EPT
CPW = EPT // 128
NCHUNK = EPAD // 128
HALF = N_NODE // 2
TRASH_A = NRATE * HALF
ARENA = 25600
ASL = ARENA // NS
DT = 50176
DSL = DT // NS
SBLK = 1000
NBLK = N_NODE // SBLK

_mesh = plsc.VectorSubcoreMesh(core_axis_name="c", subcore_axis_name="s")
_sc_params = pltpu.CompilerParams(use_tc_tiling_on_sc=False)


def _counts_body(src_h, dst_h, rate_h, g0_out, a0_out, g1_out, a1_out,
                 dsrc_out, ddst_out,
                 srcv, dstv, ratev, g20, a20, g21, a21, ds2, dd2, onesv,
                 zbuf, dsrc_sh, ddst_sh):
    c = lax.axis_index("c")
    s = lax.axis_index("s")
    wid = c * NS + s
    z16 = jnp.zeros((16,), jnp.float32)

    def fz(i, _):
        zbuf[pl.ds(i * 16, 16)] = z16
        return 0
    lax.fori_loop(0, DSL // 16, fz, 0)

    o16 = jnp.ones((16,), jnp.float32)

    def fo(i, _):
        onesv[pl.ds(i * 16, 16)] = o16
        return 0
    lax.fori_loop(0, 8, fo, 0)

    off = pl.multiple_of(s * DSL, 8)
    pltpu.sync_copy(zbuf, dsrc_sh.at[pl.ds(off, DSL)])
    pltpu.sync_copy(zbuf, ddst_sh.at[pl.ds(off, DSL)])
    plsc.subcore_barrier()

    ebase = wid * EPT
    iota16 = lax.iota(jnp.int32, 16)

    def outer(oc, _):
        base = pl.multiple_of(ebase + oc * 1024, 8)
        pltpu.sync_copy(src_h.at[pl.ds(base, 1024)], srcv)
        pltpu.sync_copy(dst_h.at[pl.ds(base, 1024)], dstv)
        pltpu.sync_copy(rate_h.at[pl.ds(base, 1024)], ratev)

        def row(jr, _):
            def col(jc, _):
                o = jr * 128 + jc * 16
                rv = ratev[pl.ds(o, 16)]
                sv = srcv[pl.ds(o, 16)]
                dv = dstv[pl.ds(o, 16)]
                half = jnp.where(dv >= HALF, 1, 0)
                dloc = dv - half * HALF
                g_raw = jnp.minimum(rv, 4) * N_NODE + sv
                a_loc = jnp.minimum(rv, 4) * HALF + dloc
                spread = TRASH_A + jnp.bitwise_and(sv + iota16, 127)
                valid = rv < NRATE
                in0 = valid & (half == 0)
                in1 = valid & (half == 1)
                cs = pl.ds(jc * 16, 16)
                g20[jr, cs] = g_raw
                a20[jr, cs] = jnp.where(in0, a_loc, spread)
                g21[jr, cs] = g_raw
                a21[jr, cs] = jnp.where(in1, a_loc, spread)
                dspread = jnp.bitwise_and(iota16 * 11 + sv + dv, 127)
                ds2[jr, cs] = jnp.where(valid, rv * N_NODE + sv,
                                        NRATE * N_NODE + dspread)
                dd2[jr, cs] = jnp.where(valid, rv * N_NODE + dv,
                                        NRATE * N_NODE + dspread)
                return 0
            lax.fori_loop(0, 8, col, 0)
            return 0
        lax.fori_loop(0, 8, row, 0)

        cb = pl.multiple_of(base // 128, 8)
        pltpu.sync_copy(g20, g0_out.at[pl.ds(cb, 8)])
        pltpu.sync_copy(a20, a0_out.at[pl.ds(cb, 8)])
        pltpu.sync_copy(g21, g1_out.at[pl.ds(cb, 8)])
        pltpu.sync_copy(a21, a1_out.at[pl.ds(cb, 8)])

        def sc8(j, _):
            pltpu.sync_copy(onesv, dsrc_sh.at[ds2.at[j]], add=True)
            pltpu.sync_copy(onesv, ddst_sh.at[dd2.at[j]], add=True)
            return 0
        lax.fori_loop(0, 8, sc8, 0)
        return 0
    lax.fori_loop(0, CPW // 8, outer, 0)

    plsc.subcore_barrier()
    coff = pl.multiple_of(c * DT + s * DSL, 8)
    pltpu.sync_copy(dsrc_sh.at[pl.ds(off, DSL)], zbuf)
    pltpu.sync_copy(zbuf, dsrc_out.at[pl.ds(coff, DSL)])
    pltpu.sync_copy(ddst_sh.at[pl.ds(off, DSL)], zbuf)
    pltpu.sync_copy(zbuf, ddst_out.at[pl.ds(coff, DSL)])


_counts_call = pl.kernel(
    _counts_body,
    out_type=[
        jax.ShapeDtypeStruct((NCHUNK, 128), jnp.int32),
        jax.ShapeDtypeStruct((NCHUNK, 128), jnp.int32),
        jax.ShapeDtypeStruct((NCHUNK, 128), jnp.int32),
        jax.ShapeDtypeStruct((NCHUNK, 128), jnp.int32),
        jax.ShapeDtypeStruct((NC * DT,), jnp.float32),
        jax.ShapeDtypeStruct((NC * DT,), jnp.float32),
    ],
    mesh=_mesh,
    compiler_params=_sc_params,
    scratch_types=[
        pltpu.VMEM((1024,), jnp.int32),
        pltpu.VMEM((1024,), jnp.int32),
        pltpu.VMEM((1024,), jnp.int32),
        pltpu.VMEM((8, 128), jnp.int32),
        pltpu.VMEM((8, 128), jnp.int32),
        pltpu.VMEM((8, 128), jnp.int32),
        pltpu.VMEM((8, 128), jnp.int32),
        pltpu.VMEM((8, 128), jnp.int32),
        pltpu.VMEM((8, 128), jnp.int32),
        pltpu.VMEM((128,), jnp.float32),
        pltpu.VMEM((DSL,), jnp.float32),
        pltpu.VMEM_SHARED((DT,), jnp.float32),
        pltpu.VMEM_SHARED((DT,), jnp.float32),
    ],
)


def _edge_body(feat_h, g0_h, a0_h, g1_h, a1_h, agg_out,
               g2, a2, rowsA, rowsB, zrow, semg, sems, agg_sh):
    c = lax.axis_index("c")
    s = lax.axis_index("s")
    wid = c * NS + s
    z16 = jnp.zeros((16,), jnp.float32)
    cbase = wid * CPW

    for p, (g_h, a_h) in enumerate(((g0_h, a0_h), (g1_h, a1_h))):
        def fz(i, _):
            zrow[i, pl.ds(0, 16)] = z16
            zrow[i, pl.ds(16, 16)] = z16
            return 0
        lax.fori_loop(0, 400, fz, 0)

        def zc(i, _):
            pltpu.sync_copy(
                zrow,
                agg_sh.at[pl.ds(pl.multiple_of(s * ASL + i * 400, 8), 400)])
            return 0
        lax.fori_loop(0, ASL // 400, zc, 0)
        plsc.subcore_barrier()

        def outer(oc, _):
            cb = pl.multiple_of(cbase + oc * 8, 8)
            pltpu.sync_copy(g_h.at[pl.ds(cb, 8)], g2)
            pltpu.sync_copy(a_h.at[pl.ds(cb, 8)], a2)

            g_pend = pltpu.async_copy(feat_h.at[g2.at[0]], rowsA, semg)
            s_pend = None
            for j in range(8):
                cur, nxt = (rowsA, rowsB) if j % 2 == 0 else (rowsB, rowsA)
                g_pend.wait()
                if s_pend is not None:
                    s_pend.wait()
                if j < 7:
                    g_pend = pltpu.async_copy(feat_h.at[g2.at[j + 1]], nxt,
                                              semg)
                s_pend = pltpu.async_copy(cur, agg_sh.at[a2.at[j]], sems,
                                          add=True)
            s_pend.wait()
            return 0
        lax.fori_loop(0, CPW // 8, outer, 0)

        plsc.subcore_barrier()

        def oc4(i, _):
            soff = pl.multiple_of(s * ASL + i * 400, 8)
            pltpu.sync_copy(agg_sh.at[pl.ds(soff, 400)], zrow)
            pltpu.sync_copy(zrow, agg_out.at[c, p, pl.ds(soff, 400)])
            return 0
        lax.fori_loop(0, ASL // 400, oc4, 0)
    plsc.subcore_barrier()


_edge_call = pl.kernel(
    _edge_body,
    out_type=[jax.ShapeDtypeStruct((NC, 2, ARENA, DOUT), jnp.float32)],
    mesh=_mesh,
    compiler_params=_sc_params,
    scratch_types=[
        pltpu.VMEM((8, 128), jnp.int32),
        pltpu.VMEM((8, 128), jnp.int32),
        pltpu.VMEM((128, DOUT), jnp.float32),
        pltpu.VMEM((128, DOUT), jnp.float32),
        pltpu.VMEM((400, DOUT), jnp.float32),
        pltpu.SemaphoreType.DMA,
        pltpu.SemaphoreType.DMA,
        pltpu.VMEM_SHARED((ARENA, DOUT), jnp.float32),
    ],
)


def _feat_body(x_ref, w_ref, dg_ref, o_ref):
    deg = dg_ref[0, 0] + dg_ref[1, 0]
    norm = lax.rsqrt(jnp.maximum(deg, 1.0))
    y = jnp.dot(x_ref[...], w_ref[0], preferred_element_type=jnp.float32)
    o_ref[...] = y * norm


_feat_call = pl.pallas_call(
    _feat_body,
    grid=(NRATE, NBLK),
    in_specs=[
        pl.BlockSpec((SBLK, DIN), lambda r, sb: (sb, 0)),
        pl.BlockSpec((1, DIN, DOUT), lambda r, sb: (r, 0, 0)),
        pl.BlockSpec((NC, 1, SBLK, 1), lambda r, sb: (0, r, sb, 0)),
    ],
    out_specs=pl.BlockSpec((SBLK, DOUT), lambda r, sb: (r * NBLK + sb, 0)),
    out_shape=jax.ShapeDtypeStruct((NRATE * N_NODE, DOUT), jnp.float32),
)


def _epi_body(arn_ref, dd_ref, b_ref, o_ref):
    hs = []
    for r in range(NRATE):
        agg = arn_ref[0, 0, r] + arn_ref[1, 0, r]
        deg = dd_ref[0, r] + dd_ref[1, r]
        norm = lax.rsqrt(jnp.maximum(deg, 1.0))
        h = agg * norm + b_ref[r][None, :]
        hs.append(jnp.where(deg > 0.0, h, 0.0))
    o_ref[...] = jnp.concatenate(hs, axis=1)


_epi_call = pl.pallas_call(
    _epi_body,
    grid=(NBLK,),
    in_specs=[
        pl.BlockSpec((NC, 1, NRATE, SBLK, DOUT),
                     lambda db: (0, db // 5, 0, db % 5, 0)),
        pl.BlockSpec((NC, NRATE, SBLK, 1), lambda db: (0, 0, db, 0)),
        pl.BlockSpec((NRATE, DOUT), lambda db: (0, 0)),
    ],
    out_specs=pl.BlockSpec((SBLK, NRATE * DOUT), lambda db: (db, 0)),
    out_shape=jax.ShapeDtypeStruct((N_NODE, NRATE * DOUT), jnp.float32),
)


@jax.jit
def kernel(x_src, x_dst, edge_index, rate, W, b):
    src = edge_index[0].astype(jnp.int32)
    dst = edge_index[1].astype(jnp.int32)
    rt = rate.astype(jnp.int32)
    pad = EPAD - NE
    zpad = jnp.zeros((pad,), jnp.int32)
    src_p = jnp.concatenate([src, zpad])
    dst_p = jnp.concatenate([dst, zpad])
    rt_p = jnp.concatenate([rt, jnp.full((pad,), NRATE, jnp.int32)])

    g0, a0, g1, a1, dsrc_p, ddst_p = _counts_call(src_p, dst_p, rt_p)
    dsrc4 = dsrc_p.reshape(NC, DT)[:, :NRATE * N_NODE].reshape(
        NC, NRATE, N_NODE, 1)
    ddst4 = ddst_p.reshape(NC, DT)[:, :NRATE * N_NODE].reshape(
        NC, NRATE, N_NODE, 1)

    feat = _feat_call(x_src, W, dsrc4)
    (arena,) = _edge_call(feat, g0, a0, g1, a1)

    arn = arena[:, :, :NRATE * HALF, :].reshape(NC, 2, NRATE, HALF, DOUT)

    return _epi_call(arn, ddst4, b)


# speedup vs baseline: 21.0174x; 1.0793x over previous
import jax
import jax.numpy as jnp
from jax import lax
from jax.experimental import pallas as pl
from jax.experimental.pallas import tpu as pltpu
from jax.experimental.pallas import tpu_sc as plsc

N_NODE = 10000
NRATE = 5
DIN = 128
DOUT = 32
NE = 320000

NC = 2
NS = 16
NW = NC * NS
EPT = 10240
EPAD = NW * EPT
CPW = EPT // 128
NCHUNK = EPAD // 128
HALF = N_NODE // 2
TRASH_A = NRATE * HALF
ARENA = 25600
ASL = ARENA // NS
DT = 50176
DSL = DT // NS
SBLK = 1000
NBLK = N_NODE // SBLK

_mesh = plsc.VectorSubcoreMesh(core_axis_name="c", subcore_axis_name="s")
_sc_params = pltpu.CompilerParams(use_tc_tiling_on_sc=False)


def _counts_body(src_h, dst_h, rate_h, g0_out, a0_out, g1_out, a1_out,
                 dsrc_out, ddst_out,
                 srcv, dstv, ratev, g20, a20, g21, a21, ds2, dd2, onesv,
                 zbuf, dsrc_sh, ddst_sh):
    c = lax.axis_index("c")
    s = lax.axis_index("s")
    wid = c * NS + s
    z16 = jnp.zeros((16,), jnp.float32)

    def fz(i, _):
        zbuf[pl.ds(i * 16, 16)] = z16
        return 0
    lax.fori_loop(0, DSL // 16, fz, 0)

    o16 = jnp.ones((16,), jnp.float32)

    def fo(i, _):
        onesv[pl.ds(i * 16, 16)] = o16
        return 0
    lax.fori_loop(0, 8, fo, 0)

    off = pl.multiple_of(s * DSL, 8)
    pltpu.sync_copy(zbuf, dsrc_sh.at[pl.ds(off, DSL)])
    pltpu.sync_copy(zbuf, ddst_sh.at[pl.ds(off, DSL)])
    plsc.subcore_barrier()

    ebase = wid * EPT
    iota16 = lax.iota(jnp.int32, 16)

    def outer(oc, _):
        base = pl.multiple_of(ebase + oc * 1024, 8)
        pltpu.sync_copy(src_h.at[pl.ds(base, 1024)], srcv)
        pltpu.sync_copy(dst_h.at[pl.ds(base, 1024)], dstv)
        pltpu.sync_copy(rate_h.at[pl.ds(base, 1024)], ratev)

        def row(jr, _):
            def col(jc, _):
                o = jr * 128 + jc * 16
                rv = ratev[pl.ds(o, 16)]
                sv = srcv[pl.ds(o, 16)]
                dv = dstv[pl.ds(o, 16)]
                half = jnp.where(dv >= HALF, 1, 0)
                dloc = dv - half * HALF
                g_raw = jnp.minimum(rv, 4) * N_NODE + sv
                a_loc = jnp.minimum(rv, 4) * HALF + dloc
                spread = TRASH_A + jnp.bitwise_and(sv + iota16, 127)
                valid = rv < NRATE
                in0 = valid & (half == 0)
                in1 = valid & (half == 1)
                cs = pl.ds(jc * 16, 16)
                g20[jr, cs] = g_raw
                a20[jr, cs] = jnp.where(in0, a_loc, spread)
                g21[jr, cs] = g_raw
                a21[jr, cs] = jnp.where(in1, a_loc, spread)
                dspread = jnp.bitwise_and(iota16 * 11 + sv + dv, 127)
                ds2[jr, cs] = jnp.where(valid, rv * N_NODE + sv,
                                        NRATE * N_NODE + dspread)
                dd2[jr, cs] = jnp.where(valid, rv * N_NODE + dv,
                                        NRATE * N_NODE + dspread)
                return 0
            lax.fori_loop(0, 8, col, 0)
            return 0
        lax.fori_loop(0, 8, row, 0)

        cb = pl.multiple_of(base // 128, 8)
        pltpu.sync_copy(g20, g0_out.at[pl.ds(cb, 8)])
        pltpu.sync_copy(a20, a0_out.at[pl.ds(cb, 8)])
        pltpu.sync_copy(g21, g1_out.at[pl.ds(cb, 8)])
        pltpu.sync_copy(a21, a1_out.at[pl.ds(cb, 8)])

        def sc8(j, _):
            pltpu.sync_copy(onesv, dsrc_sh.at[ds2.at[j]], add=True)
            pltpu.sync_copy(onesv, ddst_sh.at[dd2.at[j]], add=True)
            return 0
        lax.fori_loop(0, 8, sc8, 0)
        return 0
    lax.fori_loop(0, CPW // 8, outer, 0)

    plsc.subcore_barrier()
    coff = pl.multiple_of(c * DT + s * DSL, 8)
    pltpu.sync_copy(dsrc_sh.at[pl.ds(off, DSL)], zbuf)
    pltpu.sync_copy(zbuf, dsrc_out.at[pl.ds(coff, DSL)])
    pltpu.sync_copy(ddst_sh.at[pl.ds(off, DSL)], zbuf)
    pltpu.sync_copy(zbuf, ddst_out.at[pl.ds(coff, DSL)])


_counts_call = pl.kernel(
    _counts_body,
    out_type=[
        jax.ShapeDtypeStruct((NCHUNK, 128), jnp.int32),
        jax.ShapeDtypeStruct((NCHUNK, 128), jnp.int32),
        jax.ShapeDtypeStruct((NCHUNK, 128), jnp.int32),
        jax.ShapeDtypeStruct((NCHUNK, 128), jnp.int32),
        jax.ShapeDtypeStruct((NC * DT,), jnp.float32),
        jax.ShapeDtypeStruct((NC * DT,), jnp.float32),
    ],
    mesh=_mesh,
    compiler_params=_sc_params,
    scratch_types=[
        pltpu.VMEM((1024,), jnp.int32),
        pltpu.VMEM((1024,), jnp.int32),
        pltpu.VMEM((1024,), jnp.int32),
        pltpu.VMEM((8, 128), jnp.int32),
        pltpu.VMEM((8, 128), jnp.int32),
        pltpu.VMEM((8, 128), jnp.int32),
        pltpu.VMEM((8, 128), jnp.int32),
        pltpu.VMEM((8, 128), jnp.int32),
        pltpu.VMEM((8, 128), jnp.int32),
        pltpu.VMEM((128,), jnp.float32),
        pltpu.VMEM((DSL,), jnp.float32),
        pltpu.VMEM_SHARED((DT,), jnp.float32),
        pltpu.VMEM_SHARED((DT,), jnp.float32),
    ],
)


def _edge_body(feat_h, g0_h, a0_h, g1_h, a1_h, agg_out,
               g2, a2, r0, r1b, r2, r3, r4, r5, r6, r7, zrow,
               sg0, sg1, sg2, sg3, sg4, sg5, sg6, sg7,
               ss0, ss1, ss2, ss3, ss4, ss5, ss6, ss7, agg_sh):
    c = lax.axis_index("c")
    s = lax.axis_index("s")
    wid = c * NS + s
    z16 = jnp.zeros((16,), jnp.float32)
    cbase = wid * CPW
    rows = (r0, r1b, r2, r3, r4, r5, r6, r7)
    semg = (sg0, sg1, sg2, sg3, sg4, sg5, sg6, sg7)
    sems = (ss0, ss1, ss2, ss3, ss4, ss5, ss6, ss7)

    for p, (g_h, a_h) in enumerate(((g0_h, a0_h), (g1_h, a1_h))):
        def fz(i, _):
            zrow[i, pl.ds(0, 16)] = z16
            zrow[i, pl.ds(16, 16)] = z16
            return 0
        lax.fori_loop(0, 400, fz, 0)

        def zc(i, _):
            pltpu.sync_copy(
                zrow,
                agg_sh.at[pl.ds(pl.multiple_of(s * ASL + i * 400, 8), 400)])
            return 0
        lax.fori_loop(0, ASL // 400, zc, 0)
        plsc.subcore_barrier()

        def outer(oc, _):
            cb = pl.multiple_of(cbase + oc * 8, 8)
            pltpu.sync_copy(g_h.at[pl.ds(cb, 8)], g2)
            pltpu.sync_copy(a_h.at[pl.ds(cb, 8)], a2)

            gps = [pltpu.async_copy(feat_h.at[g2.at[j]], rows[j], semg[j])
                   for j in range(8)]
            sps = []
            for j in range(8):
                gps[j].wait()
                sps.append(pltpu.async_copy(rows[j], agg_sh.at[a2.at[j]],
                                            sems[j], add=True))
            for sp in sps:
                sp.wait()
            return 0
        lax.fori_loop(0, CPW // 8, outer, 0)

        plsc.subcore_barrier()

        def oc4(i, _):
            soff = pl.multiple_of(s * ASL + i * 400, 8)
            pltpu.sync_copy(agg_sh.at[pl.ds(soff, 400)], zrow)
            pltpu.sync_copy(zrow, agg_out.at[c, p, pl.ds(soff, 400)])
            return 0
        lax.fori_loop(0, ASL // 400, oc4, 0)
    plsc.subcore_barrier()


_edge_call = pl.kernel(
    _edge_body,
    out_type=[jax.ShapeDtypeStruct((NC, 2, ARENA, DOUT), jnp.float32)],
    mesh=_mesh,
    compiler_params=_sc_params,
    scratch_types=(
        [pltpu.VMEM((8, 128), jnp.int32)] * 2
        + [pltpu.VMEM((128, DOUT), jnp.float32)] * 8
        + [pltpu.VMEM((400, DOUT), jnp.float32)]
        + [pltpu.SemaphoreType.DMA] * 16
        + [pltpu.VMEM_SHARED((ARENA, DOUT), jnp.float32)]
    ),
)


def _feat_body(x_ref, w_ref, dg_ref, o_ref):
    deg = dg_ref[0, 0] + dg_ref[1, 0]
    norm = lax.rsqrt(jnp.maximum(deg, 1.0))
    y = jnp.dot(x_ref[...], w_ref[0], preferred_element_type=jnp.float32)
    o_ref[...] = y * norm


_feat_call = pl.pallas_call(
    _feat_body,
    grid=(NRATE, NBLK),
    in_specs=[
        pl.BlockSpec((SBLK, DIN), lambda r, sb: (sb, 0)),
        pl.BlockSpec((1, DIN, DOUT), lambda r, sb: (r, 0, 0)),
        pl.BlockSpec((NC, 1, SBLK, 1), lambda r, sb: (0, r, sb, 0)),
    ],
    out_specs=pl.BlockSpec((SBLK, DOUT), lambda r, sb: (r * NBLK + sb, 0)),
    out_shape=jax.ShapeDtypeStruct((NRATE * N_NODE, DOUT), jnp.float32),
)


def _epi_body(arn_ref, dd_ref, b_ref, o_ref):
    hs = []
    for r in range(NRATE):
        agg = arn_ref[0, 0, r] + arn_ref[1, 0, r]
        deg = dd_ref[0, r] + dd_ref[1, r]
        norm = lax.rsqrt(jnp.maximum(deg, 1.0))
        h = agg * norm + b_ref[r][None, :]
        hs.append(jnp.where(deg > 0.0, h, 0.0))
    o_ref[...] = jnp.concatenate(hs, axis=1)


_epi_call = pl.pallas_call(
    _epi_body,
    grid=(NBLK,),
    in_specs=[
        pl.BlockSpec((NC, 1, NRATE, SBLK, DOUT),
                     lambda db: (0, db // 5, 0, db % 5, 0)),
        pl.BlockSpec((NC, NRATE, SBLK, 1), lambda db: (0, 0, db, 0)),
        pl.BlockSpec((NRATE, DOUT), lambda db: (0, 0)),
    ],
    out_specs=pl.BlockSpec((SBLK, NRATE * DOUT), lambda db: (db, 0)),
    out_shape=jax.ShapeDtypeStruct((N_NODE, NRATE * DOUT), jnp.float32),
)


@jax.jit
def kernel(x_src, x_dst, edge_index, rate, W, b):
    src = edge_index[0].astype(jnp.int32)
    dst = edge_index[1].astype(jnp.int32)
    rt = rate.astype(jnp.int32)
    pad = EPAD - NE
    zpad = jnp.zeros((pad,), jnp.int32)
    src_p = jnp.concatenate([src, zpad])
    dst_p = jnp.concatenate([dst, zpad])
    rt_p = jnp.concatenate([rt, jnp.full((pad,), NRATE, jnp.int32)])

    g0, a0, g1, a1, dsrc_p, ddst_p = _counts_call(src_p, dst_p, rt_p)
    dsrc4 = dsrc_p.reshape(NC, DT)[:, :NRATE * N_NODE].reshape(
        NC, NRATE, N_NODE, 1)
    ddst4 = ddst_p.reshape(NC, DT)[:, :NRATE * N_NODE].reshape(
        NC, NRATE, N_NODE, 1)

    feat = _feat_call(x_src, W, dsrc4)
    (arena,) = _edge_call(feat, g0, a0, g1, a1)

    arn = arena[:, :, :NRATE * HALF, :].reshape(NC, 2, NRATE, HALF, DOUT)

    return _epi_call(arn, ddst4, b)


# speedup vs baseline: 30.9100x; 1.4707x over previous
import jax
import jax.numpy as jnp
from jax import lax
from jax.experimental import pallas as pl
from jax.experimental.pallas import tpu as pltpu
from jax.experimental.pallas import tpu_sc as plsc

N_NODE = 10000
NRATE = 5
DIN = 128
DOUT = 32
NE = 320000

NC = 2
NS = 16
NW = NC * NS
EPT = 10240
EPAD = NW * EPT
CPW = EPT // 128
NCHUNK = EPAD // 128
TRASH_A = NRATE * N_NODE
ARENA = 50176
ASL = ARENA // NS
ZR = 392
DT = 50176
DSL = DT // NS
SBLK = 1000
NBLK = N_NODE // SBLK

_mesh = plsc.VectorSubcoreMesh(core_axis_name="c", subcore_axis_name="s")
_sc_params = pltpu.CompilerParams(use_tc_tiling_on_sc=False)


def _counts_body(src_h, dst_h, rate_h, g_out, a_out, dsrc_out, ddst_out,
                 srcv, dstv, ratev, g2, a2, ds2, dd2, onesv,
                 zbuf, dsrc_sh, ddst_sh):
    c = lax.axis_index("c")
    s = lax.axis_index("s")
    wid = c * NS + s
    z16 = jnp.zeros((16,), jnp.float32)

    def fz(i, _):
        zbuf[pl.ds(i * 16, 16)] = z16
        return 0
    lax.fori_loop(0, DSL // 16, fz, 0)

    o16 = jnp.ones((16,), jnp.float32)

    def fo(i, _):
        onesv[pl.ds(i * 16, 16)] = o16
        return 0
    lax.fori_loop(0, 8, fo, 0)

    off = pl.multiple_of(s * DSL, 8)
    pltpu.sync_copy(zbuf, dsrc_sh.at[pl.ds(off, DSL)])
    pltpu.sync_copy(zbuf, ddst_sh.at[pl.ds(off, DSL)])
    plsc.subcore_barrier()

    ebase = wid * EPT
    iota16 = lax.iota(jnp.int32, 16)

    def outer(oc, _):
        base = pl.multiple_of(ebase + oc * 1024, 8)
        pltpu.sync_copy(src_h.at[pl.ds(base, 1024)], srcv)
        pltpu.sync_copy(dst_h.at[pl.ds(base, 1024)], dstv)
        pltpu.sync_copy(rate_h.at[pl.ds(base, 1024)], ratev)

        def row(jr, _):
            def col(jc, _):
                o = jr * 128 + jc * 16
                rv = ratev[pl.ds(o, 16)]
                sv = srcv[pl.ds(o, 16)]
                dv = dstv[pl.ds(o, 16)]
                r4 = jnp.minimum(rv, 4)
                g_raw = r4 * N_NODE + sv
                a_loc = r4 * N_NODE + dv
                spread = TRASH_A + jnp.bitwise_and(sv + iota16, 127)
                valid = rv < NRATE
                cs = pl.ds(jc * 16, 16)
                g2[jr, cs] = g_raw
                a2[jr, cs] = jnp.where(valid, a_loc, spread)
                dspread = jnp.bitwise_and(iota16 * 11 + sv + dv, 127)
                ds2[jr, cs] = jnp.where(valid, rv * N_NODE + sv,
                                        NRATE * N_NODE + dspread)
                dd2[jr, cs] = jnp.where(valid, rv * N_NODE + dv,
                                        NRATE * N_NODE + dspread)
                return 0
            lax.fori_loop(0, 8, col, 0)
            return 0
        lax.fori_loop(0, 8, row, 0)

        cb = pl.multiple_of(base // 128, 8)
        pltpu.sync_copy(g2, g_out.at[pl.ds(cb, 8)])
        pltpu.sync_copy(a2, a_out.at[pl.ds(cb, 8)])

        def sc8(j, _):
            pltpu.sync_copy(onesv, dsrc_sh.at[ds2.at[j]], add=True)
            pltpu.sync_copy(onesv, ddst_sh.at[dd2.at[j]], add=True)
            return 0
        lax.fori_loop(0, 8, sc8, 0)
        return 0
    lax.fori_loop(0, CPW // 8, outer, 0)

    plsc.subcore_barrier()
    coff = pl.multiple_of(c * DT + s * DSL, 8)
    pltpu.sync_copy(dsrc_sh.at[pl.ds(off, DSL)], zbuf)
    pltpu.sync_copy(zbuf, dsrc_out.at[pl.ds(coff, DSL)])
    pltpu.sync_copy(ddst_sh.at[pl.ds(off, DSL)], zbuf)
    pltpu.sync_copy(zbuf, ddst_out.at[pl.ds(coff, DSL)])


_counts_call = pl.kernel(
    _counts_body,
    out_type=[
        jax.ShapeDtypeStruct((NCHUNK, 128), jnp.int32),
        jax.ShapeDtypeStruct((NCHUNK, 128), jnp.int32),
        jax.ShapeDtypeStruct((NC * DT,), jnp.float32),
        jax.ShapeDtypeStruct((NC * DT,), jnp.float32),
    ],
    mesh=_mesh,
    compiler_params=_sc_params,
    scratch_types=[
        pltpu.VMEM((1024,), jnp.int32),
        pltpu.VMEM((1024,), jnp.int32),
        pltpu.VMEM((1024,), jnp.int32),
        pltpu.VMEM((8, 128), jnp.int32),
        pltpu.VMEM((8, 128), jnp.int32),
        pltpu.VMEM((8, 128), jnp.int32),
        pltpu.VMEM((8, 128), jnp.int32),
        pltpu.VMEM((128,), jnp.float32),
        pltpu.VMEM((DSL,), jnp.float32),
        pltpu.VMEM_SHARED((DT,), jnp.float32),
        pltpu.VMEM_SHARED((DT,), jnp.float32),
    ],
)


def _edge_body(feat_h, g_h, a_h, zeros_h, agg_out,
               g2, a2, r0, r1b, r2, r3, r4, r5, r6, r7, zrow,
               sg0, sg1, sg2, sg3, sg4, sg5, sg6, sg7,
               ss0, ss1, ss2, ss3, ss4, ss5, ss6, ss7, agg_sh):
    c = lax.axis_index("c")
    s = lax.axis_index("s")
    wid = c * NS + s
    cbase = wid * CPW
    rows = (r0, r1b, r2, r3, r4, r5, r6, r7)
    semg = (sg0, sg1, sg2, sg3, sg4, sg5, sg6, sg7)
    sems = (ss0, ss1, ss2, ss3, ss4, ss5, ss6, ss7)

    pltpu.sync_copy(zeros_h, zrow)

    def zc(i, _):
        pltpu.sync_copy(
            zrow,
            agg_sh.at[pl.ds(pl.multiple_of(s * ASL + i * ZR, 8), ZR)])
        return 0
    lax.fori_loop(0, ASL // ZR, zc, 0)
    plsc.subcore_barrier()

    def outer(oc, _):
        cb = pl.multiple_of(cbase + oc * 8, 8)
        pltpu.sync_copy(g_h.at[pl.ds(cb, 8)], g2)
        pltpu.sync_copy(a_h.at[pl.ds(cb, 8)], a2)

        gps = [pltpu.async_copy(feat_h.at[g2.at[j]], rows[j], semg[j])
               for j in range(8)]
        sps = []
        for j in range(8):
            gps[j].wait()
            sps.append(pltpu.async_copy(rows[j], agg_sh.at[a2.at[j]],
                                        sems[j], add=True))
        for sp in sps:
            sp.wait()
        return 0
    lax.fori_loop(0, CPW // 8, outer, 0)

    plsc.subcore_barrier()

    def oc4(i, _):
        soff = pl.multiple_of(s * ASL + i * ZR, 8)
        pltpu.sync_copy(agg_sh.at[pl.ds(soff, ZR)], zrow)
        pltpu.sync_copy(zrow, agg_out.at[c, pl.ds(soff, ZR)])
        return 0
    lax.fori_loop(0, ASL // ZR, oc4, 0)
    plsc.subcore_barrier()


_edge_call = pl.kernel(
    _edge_body,
    out_type=[jax.ShapeDtypeStruct((NC, ARENA, DOUT), jnp.bfloat16)],
    mesh=_mesh,
    compiler_params=_sc_params,
    scratch_types=(
        [pltpu.VMEM((8, 128), jnp.int32)] * 2
        + [pltpu.VMEM((128, DOUT), jnp.bfloat16)] * 8
        + [pltpu.VMEM((ZR, DOUT), jnp.bfloat16)]
        + [pltpu.SemaphoreType.DMA] * 16
        + [pltpu.VMEM_SHARED((ARENA, DOUT), jnp.bfloat16)]
    ),
)


def _feat_body(x_ref, w_ref, dg_ref, o_ref):
    deg = dg_ref[0, 0] + dg_ref[1, 0]
    norm = lax.rsqrt(jnp.maximum(deg, 1.0))
    y = jnp.dot(x_ref[...], w_ref[0], preferred_element_type=jnp.float32)
    o_ref[...] = (y * norm).astype(jnp.bfloat16)


_feat_call = pl.pallas_call(
    _feat_body,
    grid=(NRATE, NBLK),
    in_specs=[
        pl.BlockSpec((SBLK, DIN), lambda r, sb: (sb, 0)),
        pl.BlockSpec((1, DIN, DOUT), lambda r, sb: (r, 0, 0)),
        pl.BlockSpec((NC, 1, SBLK, 1), lambda r, sb: (0, r, sb, 0)),
    ],
    out_specs=pl.BlockSpec((SBLK, DOUT), lambda r, sb: (r * NBLK + sb, 0)),
    out_shape=jax.ShapeDtypeStruct((NRATE * N_NODE, DOUT), jnp.bfloat16),
)


def _epi_body(arn_ref, dd_ref, b_ref, o_ref):
    hs = []
    for r in range(NRATE):
        agg = (arn_ref[0, r].astype(jnp.float32)
               + arn_ref[1, r].astype(jnp.float32))
        deg = dd_ref[0, r] + dd_ref[1, r]
        norm = lax.rsqrt(jnp.maximum(deg, 1.0))
        h = agg * norm + b_ref[r][None, :]
        hs.append(jnp.where(deg > 0.0, h, 0.0))
    o_ref[...] = jnp.concatenate(hs, axis=1)


_epi_call = pl.pallas_call(
    _epi_body,
    grid=(NBLK,),
    in_specs=[
        pl.BlockSpec((NC, NRATE, SBLK, DOUT), lambda db: (0, 0, db, 0)),
        pl.BlockSpec((NC, NRATE, SBLK, 1), lambda db: (0, 0, db, 0)),
        pl.BlockSpec((NRATE, DOUT), lambda db: (0, 0)),
    ],
    out_specs=pl.BlockSpec((SBLK, NRATE * DOUT), lambda db: (db, 0)),
    out_shape=jax.ShapeDtypeStruct((N_NODE, NRATE * DOUT), jnp.float32),
)


@jax.jit
def kernel(x_src, x_dst, edge_index, rate, W, b):
    src = edge_index[0].astype(jnp.int32)
    dst = edge_index[1].astype(jnp.int32)
    rt = rate.astype(jnp.int32)
    pad = EPAD - NE
    zpad = jnp.zeros((pad,), jnp.int32)
    src_p = jnp.concatenate([src, zpad])
    dst_p = jnp.concatenate([dst, zpad])
    rt_p = jnp.concatenate([rt, jnp.full((pad,), NRATE, jnp.int32)])

    g, a, dsrc_p, ddst_p = _counts_call(src_p, dst_p, rt_p)
    dsrc4 = dsrc_p.reshape(NC, DT)[:, :NRATE * N_NODE].reshape(
        NC, NRATE, N_NODE, 1)
    ddst4 = ddst_p.reshape(NC, DT)[:, :NRATE * N_NODE].reshape(
        NC, NRATE, N_NODE, 1)

    feat = _feat_call(x_src, W, dsrc4)
    zeros_bf = jnp.zeros((ZR, DOUT), jnp.bfloat16)
    (arena,) = _edge_call(feat, g, a, zeros_bf)

    arn = arena[:, :NRATE * N_NODE, :].reshape(NC, NRATE, N_NODE, DOUT)

    return _epi_call(arn, ddst4, b)


# speedup vs baseline: 32.2800x; 1.0443x over previous
import jax
import jax.numpy as jnp
from jax import lax
from jax.experimental import pallas as pl
from jax.experimental.pallas import tpu as pltpu
from jax.experimental.pallas import tpu_sc as plsc

N_NODE = 10000
NRATE = 5
DIN = 128
DOUT = 32
NE = 320000

NC = 2
NS = 16
NW = NC * NS
EPT = 10240
EPAD = NW * EPT
CPW = EPT // 128
NCHUNK = EPAD // 128
TRASH_A = NRATE * N_NODE
ARENA = 50176
ASL = ARENA // NS
ZR = 392
DT = 50176
DSL = DT // NS
SBLK = 1000
NBLK = N_NODE // SBLK

_mesh = plsc.VectorSubcoreMesh(core_axis_name="c", subcore_axis_name="s")
_sc_params = pltpu.CompilerParams(use_tc_tiling_on_sc=False)


def _counts_body(src_h, dst_h, rate_h, g_out, a_out, dsrc_out, ddst_out,
                 srcv, dstv, ratev, g2, a2, ds2, dd2, onesv,
                 zbuf, dsrc_sh, ddst_sh):
    c = lax.axis_index("c")
    s = lax.axis_index("s")
    wid = c * NS + s
    z16 = jnp.zeros((16,), jnp.float32)

    def fz(i, _):
        zbuf[pl.ds(i * 16, 16)] = z16
        return 0
    lax.fori_loop(0, DSL // 16, fz, 0)

    o16 = jnp.ones((16,), jnp.float32)

    def fo(i, _):
        onesv[pl.ds(i * 16, 16)] = o16
        return 0
    lax.fori_loop(0, 8, fo, 0)

    off = pl.multiple_of(s * DSL, 8)
    pltpu.sync_copy(zbuf, dsrc_sh.at[pl.ds(off, DSL)])
    pltpu.sync_copy(zbuf, ddst_sh.at[pl.ds(off, DSL)])
    plsc.subcore_barrier()

    ebase = wid * EPT
    iota16 = lax.iota(jnp.int32, 16)

    def outer(oc, _):
        base = pl.multiple_of(ebase + oc * 1024, 8)
        pltpu.sync_copy(src_h.at[pl.ds(base, 1024)], srcv)
        pltpu.sync_copy(dst_h.at[pl.ds(base, 1024)], dstv)
        pltpu.sync_copy(rate_h.at[pl.ds(base, 1024)], ratev)

        def row(jr, _):
            def col(jc, _):
                o = jr * 128 + jc * 16
                rv = ratev[pl.ds(o, 16)]
                sv = srcv[pl.ds(o, 16)]
                dv = dstv[pl.ds(o, 16)]
                r4 = jnp.minimum(rv, 4)
                g_raw = r4 * N_NODE + sv
                a_loc = r4 * N_NODE + dv
                spread = TRASH_A + jnp.bitwise_and(sv + iota16, 127)
                valid = rv < NRATE
                cs = pl.ds(jc * 16, 16)
                g2[jr, cs] = g_raw
                a2[jr, cs] = jnp.where(valid, a_loc, spread)
                dspread = jnp.bitwise_and(iota16 * 11 + sv + dv, 127)
                ds2[jr, cs] = jnp.where(valid, rv * N_NODE + sv,
                                        NRATE * N_NODE + dspread)
                dd2[jr, cs] = jnp.where(valid, rv * N_NODE + dv,
                                        NRATE * N_NODE + dspread)
                return 0
            lax.fori_loop(0, 8, col, 0)
            return 0
        lax.fori_loop(0, 8, row, 0)

        cb = pl.multiple_of(base // 128, 8)
        pltpu.sync_copy(g2, g_out.at[pl.ds(cb, 8)])
        pltpu.sync_copy(a2, a_out.at[pl.ds(cb, 8)])

        def sc8(j, _):
            pltpu.sync_copy(onesv, dsrc_sh.at[ds2.at[j]], add=True)
            pltpu.sync_copy(onesv, ddst_sh.at[dd2.at[j]], add=True)
            return 0
        lax.fori_loop(0, 8, sc8, 0)
        return 0
    lax.fori_loop(0, CPW // 8, outer, 0)

    plsc.subcore_barrier()
    coff = pl.multiple_of(c * DT + s * DSL, 8)
    pltpu.sync_copy(dsrc_sh.at[pl.ds(off, DSL)], zbuf)
    pltpu.sync_copy(zbuf, dsrc_out.at[pl.ds(coff, DSL)])
    pltpu.sync_copy(ddst_sh.at[pl.ds(off, DSL)], zbuf)
    pltpu.sync_copy(zbuf, ddst_out.at[pl.ds(coff, DSL)])


_counts_call = pl.kernel(
    _counts_body,
    out_type=[
        jax.ShapeDtypeStruct((NCHUNK, 128), jnp.int32),
        jax.ShapeDtypeStruct((NCHUNK, 128), jnp.int32),
        jax.ShapeDtypeStruct((NC * DT,), jnp.float32),
        jax.ShapeDtypeStruct((NC * DT,), jnp.float32),
    ],
    mesh=_mesh,
    compiler_params=_sc_params,
    scratch_types=[
        pltpu.VMEM((1024,), jnp.int32),
        pltpu.VMEM((1024,), jnp.int32),
        pltpu.VMEM((1024,), jnp.int32),
        pltpu.VMEM((8, 128), jnp.int32),
        pltpu.VMEM((8, 128), jnp.int32),
        pltpu.VMEM((8, 128), jnp.int32),
        pltpu.VMEM((8, 128), jnp.int32),
        pltpu.VMEM((128,), jnp.float32),
        pltpu.VMEM((DSL,), jnp.float32),
        pltpu.VMEM_SHARED((DT,), jnp.float32),
        pltpu.VMEM_SHARED((DT,), jnp.float32),
    ],
)


def _edge_body(feat_h, g_h, a_h, zeros_h, agg_out, g2, a2, *rest):
    rows = rest[:8]
    zrow = rest[8]
    semg = rest[9:17]
    sems = rest[17:25]
    agg_sh = rest[25]
    c = lax.axis_index("c")
    s = lax.axis_index("s")
    wid = c * NS + s
    cbase = wid * CPW

    pltpu.sync_copy(zeros_h, zrow)

    def zc(i, _):
        pltpu.sync_copy(
            zrow,
            agg_sh.at[pl.ds(pl.multiple_of(s * ASL + i * ZR, 8), ZR)])
        return 0
    lax.fori_loop(0, ASL // ZR, zc, 0)
    plsc.subcore_barrier()

    def outer(oc, _):
        cb = pl.multiple_of(cbase + oc * 8, 8)
        pltpu.sync_copy(g_h.at[pl.ds(cb, 8)], g2)
        pltpu.sync_copy(a_h.at[pl.ds(cb, 8)], a2)

        gps = [pltpu.async_copy(feat_h.at[g2.at[j]], rows[j], semg[j])
               for j in range(8)]
        sps = []
        for j in range(8):
            gps[j].wait()
            sps.append(pltpu.async_copy(rows[j], agg_sh.at[a2.at[j]],
                                        sems[j], add=True))
        for sp in sps:
            sp.wait()
        return 0
    lax.fori_loop(0, CPW // 8, outer, 0)

    plsc.subcore_barrier()

    def oc4(i, _):
        soff = pl.multiple_of(s * ASL + i * ZR, 8)
        pltpu.sync_copy(agg_sh.at[pl.ds(soff, ZR)], zrow)
        pltpu.sync_copy(zrow, agg_out.at[c, pl.ds(soff, ZR)])
        return 0
    lax.fori_loop(0, ASL // ZR, oc4, 0)
    plsc.subcore_barrier()


_edge_call = pl.kernel(
    _edge_body,
    out_type=[jax.ShapeDtypeStruct((NC, ARENA, DOUT), jnp.bfloat16)],
    mesh=_mesh,
    compiler_params=_sc_params,
    scratch_types=(
        [pltpu.VMEM((8, 128), jnp.int32)] * 2
        + [pltpu.VMEM((128, DOUT), jnp.bfloat16)] * 8
        + [pltpu.VMEM((ZR, DOUT), jnp.bfloat16)]
        + [pltpu.SemaphoreType.DMA] * 16
        + [pltpu.VMEM_SHARED((ARENA, DOUT), jnp.bfloat16)]
    ),
)


def _feat_body(x_ref, w_ref, dg_ref, o_ref):
    deg = dg_ref[0, 0] + dg_ref[1, 0]
    norm = lax.rsqrt(jnp.maximum(deg, 1.0))
    y = jnp.dot(x_ref[...], w_ref[0], preferred_element_type=jnp.float32)
    o_ref[...] = (y * norm).astype(jnp.bfloat16)


_feat_call = pl.pallas_call(
    _feat_body,
    grid=(NRATE, NBLK),
    in_specs=[
        pl.BlockSpec((SBLK, DIN), lambda r, sb: (sb, 0)),
        pl.BlockSpec((1, DIN, DOUT), lambda r, sb: (r, 0, 0)),
        pl.BlockSpec((NC, 1, SBLK, 1), lambda r, sb: (0, r, sb, 0)),
    ],
    out_specs=pl.BlockSpec((SBLK, DOUT), lambda r, sb: (r * NBLK + sb, 0)),
    out_shape=jax.ShapeDtypeStruct((NRATE * N_NODE, DOUT), jnp.bfloat16),
)


def _epi_body(a0_ref, a1_ref, a2_ref, a3_ref, a4_ref, dd_ref, b_ref, o_ref):
    hs = []
    for r, ar in enumerate((a0_ref, a1_ref, a2_ref, a3_ref, a4_ref)):
        agg = (ar[0].astype(jnp.float32)
               + ar[1].astype(jnp.float32))
        deg = dd_ref[0, r] + dd_ref[1, r]
        norm = lax.rsqrt(jnp.maximum(deg, 1.0))
        h = agg * norm + b_ref[r][None, :]
        hs.append(jnp.where(deg > 0.0, h, 0.0))
    o_ref[...] = jnp.concatenate(hs, axis=1)


_epi_call = pl.pallas_call(
    _epi_body,
    grid=(NBLK,),
    in_specs=[
        pl.BlockSpec((NC, SBLK, DOUT), lambda db, r=r: (0, r * NBLK + db, 0))
        for r in range(NRATE)
    ] + [
        pl.BlockSpec((NC, NRATE, SBLK, 1), lambda db: (0, 0, db, 0)),
        pl.BlockSpec((NRATE, DOUT), lambda db: (0, 0)),
    ],
    out_specs=pl.BlockSpec((SBLK, NRATE * DOUT), lambda db: (db, 0)),
    out_shape=jax.ShapeDtypeStruct((N_NODE, NRATE * DOUT), jnp.float32),
)


@jax.jit
def kernel(x_src, x_dst, edge_index, rate, W, b):
    src = edge_index[0].astype(jnp.int32)
    dst = edge_index[1].astype(jnp.int32)
    rt = rate.astype(jnp.int32)
    pad = EPAD - NE
    zpad = jnp.zeros((pad,), jnp.int32)
    src_p = jnp.concatenate([src, zpad])
    dst_p = jnp.concatenate([dst, zpad])
    rt_p = jnp.concatenate([rt, jnp.full((pad,), NRATE, jnp.int32)])

    g, a, dsrc_p, ddst_p = _counts_call(src_p, dst_p, rt_p)
    dsrc4 = dsrc_p.reshape(NC, DT)[:, :NRATE * N_NODE].reshape(
        NC, NRATE, N_NODE, 1)
    ddst4 = ddst_p.reshape(NC, DT)[:, :NRATE * N_NODE].reshape(
        NC, NRATE, N_NODE, 1)

    feat = _feat_call(x_src, W, dsrc4)
    zeros_bf = jnp.zeros((ZR, DOUT), jnp.bfloat16)
    (arena,) = _edge_call(feat, g, a, zeros_bf)

    return _epi_call(arena, arena, arena, arena, arena, ddst4, b)
